# Initial kernel scaffold; baseline (speedup 1.0000x reference)
import jax
import jax.numpy as jnp
from jax.experimental import pallas as pl


def kernel(x, edge_index, edge_attr, adj, W1, a_src1, a_dst1, W2, a_src2, a_dst2, W3, a_src3, a_dst3):
    raise NotImplementedError("write your pallas kernel here")


# speedup vs baseline: 23.8116x; 23.8116x over previous
import jax
import jax.numpy as jnp
from jax import lax
from jax.experimental import pallas as pl
from jax.experimental.pallas import tpu as pltpu
from jax.experimental.pallas import tpu_sc as plsc

F32 = jnp.float32
I32 = jnp.int32

N = 10000
E = 160000
NC, NS, LANES = 2, 16, 16
NW = NC * NS
CSZ = 128
CH = 40
E_PAD = NW * CSZ * CH
SRCW = 48
DSTW = 16
OUTW = 40
ACC_PER_SUB = OUTW * N // NS


def _esc_body(a0, a1, a2, a3, o):
    rows = lax.broadcasted_iota(I32, o.shape, 0)
    cols = lax.broadcasted_iota(I32, o.shape, 1)
    eid = rows * 128 + cols
    s = a0[...] + a1[...] + a2[...] + a3[...]
    o[...] = jnp.where(eid < E, s, jnp.full_like(s, -1e30))


def _compute_esc(ea_cols):
    shp = ea_cols[0].shape
    return pl.pallas_call(
        _esc_body,
        out_shape=jax.ShapeDtypeStruct(shp, F32),
    )(*ea_cols)


def _node1_body(x_ref, w_ref, As_ref, Ad_ref, tsrc_ref, tdst_ref):
    hT = lax.dot_general(w_ref[...], x_ref[...], (((0,), (1,)), ((), ())),
                         preferred_element_type=F32)
    asrcT = lax.dot_general(As_ref[...], hT, (((0,), (0,)), ((), ())),
                            preferred_element_type=F32)
    adstT = lax.dot_general(Ad_ref[...], hT, (((0,), (0,)), ((), ())),
                            preferred_element_type=F32)
    n = hT.shape[1]
    tsrc_ref[...] = jnp.concatenate(
        [hT, asrcT, jnp.zeros((SRCW - 35, n), F32)], axis=0)
    tdst_ref[...] = jnp.concatenate(
        [adstT, jnp.zeros((DSTW - 5, n), F32)], axis=0)


def _node_tables1(x, W1, As, Ad):
    return pl.pallas_call(
        _node1_body,
        out_shape=[
            jax.ShapeDtypeStruct((SRCW, N), F32),
            jax.ShapeDtypeStruct((DSTW, N), F32),
        ],
    )(x, W1, As, Ad)


def _sc_edge_pass(M, H):
    mesh = plsc.VectorSubcoreMesh(
        core_axis_name="c", subcore_axis_name="s",
        num_cores=NC, num_subcores=NS)

    def body(tsrc, tdst, srci, dsti, esc, zrows, out,
             idxs_v, idxd_v, esc_v, gs_v, gd_v, ob_v, stage_v, acc_sh, sem):
        c = lax.axis_index("c")
        s_ = lax.axis_index("s")
        tid = c * NS + s_
        pltpu.sync_copy(zrows, stage_v)
        pltpu.sync_copy(stage_v, acc_sh.at[pl.ds(s_ * ACC_PER_SUB,
                                                 ACC_PER_SUB)])
        plsc.subcore_barrier()

        base = tid * (CH * CSZ)
        oc = M // H

        def chunk(ci, carry):
            off = base + ci * CSZ
            pltpu.sync_copy(srci.at[pl.ds(off, CSZ)], idxs_v)
            pltpu.sync_copy(dsti.at[pl.ds(off, CSZ)], idxd_v)
            pltpu.sync_copy(esc.at[pl.ds(off, CSZ)], esc_v)
            gathers = []
            for f in range(M + H):
                gathers.append(pltpu.async_copy(
                    tsrc.at[pl.ds(f * N, N)].at[idxs_v],
                    gs_v.at[pl.ds(f * CSZ, CSZ)], sem))
            for f in range(H):
                gathers.append(pltpu.async_copy(
                    tdst.at[pl.ds(f * N, N)].at[idxd_v],
                    gd_v.at[pl.ds(f * CSZ, CSZ)], sem))
            for g_ in gathers:
                g_.wait()
            for g in range(CSZ // LANES):
                o_ = g * LANES
                ev = esc_v[pl.ds(o_, LANES)]
                exs = []
                for h in range(H):
                    a_s = gs_v[pl.ds((M + h) * CSZ + o_, LANES)]
                    a_d = gd_v[pl.ds(h * CSZ + o_, LANES)]
                    l = a_s + a_d + ev
                    l = jnp.where(l >= 0, l, l * 0.2)
                    ex = jnp.exp(l)
                    ob_v[pl.ds((M + h) * CSZ + o_, LANES)] = ex
                    exs.append(ex)
                for m in range(M):
                    hv = gs_v[pl.ds(m * CSZ + o_, LANES)]
                    ob_v[pl.ds(m * CSZ + o_, LANES)] = hv * exs[m // oc]
            scatters = []
            for f in range(M + H):
                scatters.append(pltpu.async_copy(
                    ob_v.at[pl.ds(f * CSZ, CSZ)],
                    acc_sh.at[pl.ds(f * N, N)].at[idxd_v], sem, add=True))
            for s2 in scatters:
                s2.wait()
            return carry

        lax.fori_loop(0, CH, chunk, 0)
        plsc.subcore_barrier()
        pltpu.sync_copy(
            acc_sh.at[pl.ds(s_ * ACC_PER_SUB, ACC_PER_SUB)], stage_v)
        pltpu.sync_copy(
            stage_v,
            out.at[pl.ds(c * (OUTW * N) + s_ * ACC_PER_SUB, ACC_PER_SUB)])

    return pl.kernel(
        body,
        out_type=jax.ShapeDtypeStruct((NC * OUTW * N,), F32),
        mesh=mesh,
        scratch_types=[
            pltpu.VMEM((CSZ,), I32),
            pltpu.VMEM((CSZ,), I32),
            pltpu.VMEM((CSZ,), F32),
            pltpu.VMEM((SRCW * CSZ,), F32),
            pltpu.VMEM((DSTW * CSZ,), F32),
            pltpu.VMEM((OUTW * CSZ,), F32),
            pltpu.VMEM((ACC_PER_SUB,), F32),
            pltpu.VMEM_SHARED((OUTW * N,), F32),
            pltpu.SemaphoreType.DMA,
        ],
    )


def _comb2_body(p_ref, w2_ref, r1_ref, a2s_ref, a2d_ref,
                x1_ref, tsrc_ref, tdst_ref):
    numT = p_ref[0, 0:30, :] + p_ref[1, 0:30, :]
    denT = p_ref[0, 30:35, :] + p_ref[1, 30:35, :]
    denbT = lax.dot_general(r1_ref[...], denT, (((0,), (0,)), ((), ())),
                            preferred_element_type=F32)
    x1T = numT / (denbT + 1e-16)
    x1_ref[...] = x1T
    h2T = lax.dot_general(w2_ref[...], x1T, (((0,), (0,)), ((), ())),
                          preferred_element_type=F32)
    asrcT = lax.dot_general(a2s_ref[...], h2T, (((1,), (0,)), ((), ())),
                            preferred_element_type=F32)
    adstT = lax.dot_general(a2d_ref[...], h2T, (((1,), (0,)), ((), ())),
                            preferred_element_type=F32)
    n = h2T.shape[1]
    tsrc_ref[...] = jnp.concatenate(
        [h2T, asrcT, jnp.zeros((SRCW - 33, n), F32)], axis=0)
    tdst_ref[...] = jnp.concatenate(
        [adstT, jnp.zeros((DSTW - 1, n), F32)], axis=0)


def _combine_and_tables2(p1, W2, R1, a_src2, a_dst2):
    return pl.pallas_call(
        _comb2_body,
        out_shape=[
            jax.ShapeDtypeStruct((30, N), F32),
            jax.ShapeDtypeStruct((SRCW, N), F32),
            jax.ShapeDtypeStruct((DSTW, N), F32),
        ],
    )(p1, W2, R1, a_src2, a_dst2)


def _softmax32T(p_ref):
    numT = p_ref[0, 0:32, :] + p_ref[1, 0:32, :]
    denT = p_ref[0, 32:33, :] + p_ref[1, 32:33, :]
    sp = numT / (denT + 1e-16)
    m = jnp.max(sp, axis=0, keepdims=True)
    ex = jnp.exp(sp - m)
    return ex / jnp.sum(ex, axis=0, keepdims=True)


BI = 200
NII = N // BI


def _adj_body(p2_ref, adj_ref, y_ref, ssq_ref, s_scr, acc_ref):
    i = pl.program_id(0)

    @pl.when(i == 0)
    def _():
        s_scr[...] = _softmax32T(p2_ref)

    a = adj_ref[...]
    y_ref[...] = lax.dot_general(a, s_scr[...], (((1,), (1,)), ((), ())),
                                 preferred_element_type=F32)

    part = jnp.sum(a * a)

    @pl.when(i == 0)
    def _():
        acc_ref[0, 0] = part

    @pl.when(i != 0)
    def _():
        acc_ref[0, 0] = acc_ref[0, 0] + part

    @pl.when(i == NII - 1)
    def _():
        ssq_ref[0, 0] = acc_ref[0, 0]


def _adj_pass(p2, adj):
    return pl.pallas_call(
        _adj_body,
        grid=(NII,),
        in_specs=[
            pl.BlockSpec((2, OUTW, N), lambda i: (0, 0, 0)),
            pl.BlockSpec((BI, N), lambda i: (i, 0)),
        ],
        out_specs=[
            pl.BlockSpec((BI, 32), lambda i: (i, 0)),
            pl.BlockSpec(memory_space=pltpu.SMEM),
        ],
        out_shape=[
            jax.ShapeDtypeStruct((N, 32), F32),
            jax.ShapeDtypeStruct((1, 1), F32),
        ],
        scratch_shapes=[
            pltpu.VMEM((32, N), F32),
            pltpu.SMEM((1, 1), F32),
        ],
    )(p2, adj)


def _tail_body(p2_ref, y_ref, x1_ref, ssq_ref, w3_ref, a3s_ref, a3d_ref,
               x3_ref, reg_ref):
    sT = _softmax32T(p2_ref)
    cT = (((1,), (1,)), ((), ()))
    c0 = (((0,), (0,)), ((), ()))
    x2 = lax.dot_general(sT, x1_ref[...], cT,
                         preferred_element_type=F32)
    adjn = lax.dot_general(sT, y_ref[...], (((1,), (0,)), ((), ())),
                           preferred_element_type=F32)
    gmat = lax.dot_general(sT, sT, cT,
                           preferred_element_type=F32)
    eye = jnp.where(
        lax.broadcasted_iota(I32, (32, 32), 0)
        == lax.broadcasted_iota(I32, (32, 32), 1),
        jnp.full((32, 32), 1.0, F32), jnp.zeros((32, 32), F32))
    tr1 = jnp.sum(adjn * eye)
    reg1 = jnp.sqrt(ssq_ref[0, 0] - 2.0 * tr1 + jnp.sum(gmat * gmat)) \
        / float(N * N)
    h3 = jnp.dot(x2, w3_ref[...], preferred_element_type=F32)
    av = lax.dot_general(h3, a3s_ref[...], cT,
                         preferred_element_type=F32)
    dv = lax.dot_general(a3d_ref[...], h3, cT,
                         preferred_element_type=F32)
    logit = av + dv + adjn
    logit = jnp.where(logit >= 0, logit, logit * 0.2)
    m0 = jnp.max(logit, axis=0, keepdims=True)
    exl = jnp.exp(logit - m0)
    alpha = exl / (jnp.sum(exl, axis=0, keepdims=True) + 1e-16)
    s2p = lax.dot_general(alpha, h3, c0,
                          preferred_element_type=F32)
    m1 = jnp.max(s2p, axis=-1, keepdims=True)
    e1 = jnp.exp(s2p - m1)
    s2 = e1 / jnp.sum(e1, axis=-1, keepdims=True)
    x3 = lax.dot_general(s2, x2, c0, preferred_element_type=F32)
    sst = lax.dot_general(s2, s2, cT, preferred_element_type=F32)
    reg2 = jnp.sqrt(jnp.sum((adjn - sst) ** 2)) / 1024.0
    x3_ref[...] = x3
    reg_ref[0, 0] = reg1 * 10.0 + reg2 * 0.1


def _tail(p2, y, x1T, ssq, W3, a3s, a3d):
    return pl.pallas_call(
        _tail_body,
        in_specs=[
            pl.BlockSpec((2, OUTW, N), lambda: (0, 0, 0)),
            pl.BlockSpec((N, 32), lambda: (0, 0)),
            pl.BlockSpec((30, N), lambda: (0, 0)),
            pl.BlockSpec(memory_space=pltpu.SMEM),
            pl.BlockSpec((30, 4), lambda: (0, 0)),
            pl.BlockSpec((1, 4), lambda: (0, 0)),
            pl.BlockSpec((1, 4), lambda: (0, 0)),
        ],
        out_specs=[
            pl.BlockSpec((4, 30), lambda: (0, 0)),
            pl.BlockSpec(memory_space=pltpu.SMEM),
        ],
        out_shape=[
            jax.ShapeDtypeStruct((4, 30), F32),
            jax.ShapeDtypeStruct((1, 1), F32),
        ],
    )(p2, y, x1T, ssq, W3, a3s, a3d)


def kernel(x, edge_index, edge_attr, adj, W1, a_src1, a_dst1,
           W2, a_src2, a_dst2, W3, a_src3, a_dst3):
    ei = edge_index.astype(I32)
    src = jnp.pad(ei[0], (0, E_PAD - E))
    dst = jnp.pad(ei[1], (0, E_PAD - E))
    eap = jnp.pad(edge_attr, ((0, E_PAD - E), (0, 0)))
    ea_cols = [eap[:, j].reshape(E_PAD // 128, 128) for j in range(4)]

    As1 = (jnp.eye(5, dtype=F32)[:, None, :]
           * a_src1[:, :, None]).reshape(30, 5)
    Ad1 = (jnp.eye(5, dtype=F32)[:, None, :]
           * a_dst1[:, :, None]).reshape(30, 5)
    R1 = jnp.repeat(jnp.eye(5, dtype=F32), 6, axis=1)
    zrows = jnp.zeros((ACC_PER_SUB,), F32)

    esc = _compute_esc(ea_cols).reshape(E_PAD)
    t1s, t1d = _node_tables1(x, W1, As1, Ad1)
    p1 = _sc_edge_pass(30, 5)(
        t1s.reshape(SRCW * N), t1d.reshape(DSTW * N), src, dst, esc, zrows)
    x1T, t2s, t2d = _combine_and_tables2(
        p1.reshape(NC, OUTW, N), W2, R1, a_src2, a_dst2)
    p2 = _sc_edge_pass(32, 1)(
        t2s.reshape(SRCW * N), t2d.reshape(DSTW * N), src, dst, esc, zrows)
    p2r = p2.reshape(NC, OUTW, N)
    y, ssq = _adj_pass(p2r, adj)
    x3, reg = _tail(p2r, y, x1T, ssq, W3, a_src3, a_dst3)
    return (x3, reg[0, 0])


# speedup vs baseline: 30.2161x; 1.2690x over previous
import jax
import jax.numpy as jnp
from jax import lax
from jax.experimental import pallas as pl
from jax.experimental.pallas import tpu as pltpu
from jax.experimental.pallas import tpu_sc as plsc

F32 = jnp.float32
I32 = jnp.int32

N = 10000
E = 160000
NC, NS, LANES = 2, 16, 16
NW = NC * NS
CSZ = 128
CH = 40
E_PAD = NW * CSZ * CH
SRCW = 48
DSTW = 16
OUTW = 40
ACC_PER_SUB = OUTW * N // NS


def _esc_body(a0, a1, a2, a3, o):
    rows = lax.broadcasted_iota(I32, o.shape, 0)
    cols = lax.broadcasted_iota(I32, o.shape, 1)
    eid = rows * 128 + cols
    s = a0[...] + a1[...] + a2[...] + a3[...]
    o[...] = jnp.where(eid < E, s, jnp.full_like(s, -1e30))


def _compute_esc(ea_cols):
    shp = ea_cols[0].shape
    return pl.pallas_call(
        _esc_body,
        out_shape=jax.ShapeDtypeStruct(shp, F32),
    )(*ea_cols)


def _node1_body(x_ref, w_ref, As_ref, Ad_ref, tsrc_ref, tdst_ref):
    hT = lax.dot_general(w_ref[...], x_ref[...], (((0,), (1,)), ((), ())),
                         preferred_element_type=F32)
    asrcT = lax.dot_general(As_ref[...], hT, (((0,), (0,)), ((), ())),
                            preferred_element_type=F32)
    adstT = lax.dot_general(Ad_ref[...], hT, (((0,), (0,)), ((), ())),
                            preferred_element_type=F32)
    n = hT.shape[1]
    tsrc_ref[...] = jnp.concatenate(
        [hT, asrcT, jnp.zeros((SRCW - 35, n), F32)], axis=0)
    tdst_ref[...] = jnp.concatenate(
        [adstT, jnp.zeros((DSTW - 5, n), F32)], axis=0)


def _node_tables1(x, W1, As, Ad):
    return pl.pallas_call(
        _node1_body,
        out_shape=[
            jax.ShapeDtypeStruct((SRCW, N), F32),
            jax.ShapeDtypeStruct((DSTW, N), F32),
        ],
    )(x, W1, As, Ad)


def _sc_edge_pass(M, H):
    mesh = plsc.VectorSubcoreMesh(
        core_axis_name="c", subcore_axis_name="s",
        num_cores=NC, num_subcores=NS)

    def body(tsrc, tdst, src2, dst2, esc2, zrows, out,
             sidx, didx, escA, escB, gsA, gsB, gdA, gdB, obA, obB,
             stage_v, acc_sh, semga, semgb, semsa, semsb):
        c = lax.axis_index("c")
        s_ = lax.axis_index("s")
        tid = c * NS + s_
        pltpu.sync_copy(src2.at[pl.ds(tid * CH, CH)], sidx)
        pltpu.sync_copy(dst2.at[pl.ds(tid * CH, CH)], didx)
        pltpu.sync_copy(zrows, stage_v)
        pltpu.sync_copy(stage_v, acc_sh.at[pl.ds(s_ * ACC_PER_SUB,
                                                 ACC_PER_SUB)])
        plsc.subcore_barrier()

        oc = M // H

        def issue_gathers(ci, esc_b, gs_b, gd_b, sem):
            pltpu.async_copy(esc2.at[tid * CH + ci], esc_b, sem)
            for f in range(M + H):
                pltpu.async_copy(
                    tsrc.at[pl.ds(f * N, N)].at[sidx.at[ci]],
                    gs_b.at[pl.ds(f * CSZ, CSZ)], sem)
            for f in range(H):
                pltpu.async_copy(
                    tdst.at[pl.ds(f * N, N)].at[didx.at[ci]],
                    gd_b.at[pl.ds(f * CSZ, CSZ)], sem)

        def drain_gathers(esc_b, gs_b, gd_b, sem):
            pltpu.make_async_copy(esc2.at[0], esc_b, sem).wait()
            for f in range(M + H):
                pltpu.make_async_copy(
                    tsrc.at[pl.ds(f * N, N)].at[sidx.at[0]],
                    gs_b.at[pl.ds(f * CSZ, CSZ)], sem).wait()
            for f in range(H):
                pltpu.make_async_copy(
                    tdst.at[pl.ds(f * N, N)].at[didx.at[0]],
                    gd_b.at[pl.ds(f * CSZ, CSZ)], sem).wait()

        def compute(esc_b, gs_b, gd_b, ob_b):
            for g in range(CSZ // LANES):
                o_ = g * LANES
                ev = esc_b[pl.ds(o_, LANES)]
                exs = []
                for h in range(H):
                    a_s = gs_b[pl.ds((M + h) * CSZ + o_, LANES)]
                    a_d = gd_b[pl.ds(h * CSZ + o_, LANES)]
                    l = a_s + a_d + ev
                    l = jnp.where(l >= 0, l, l * 0.2)
                    ex = jnp.exp(l)
                    ob_b[pl.ds((M + h) * CSZ + o_, LANES)] = ex
                    exs.append(ex)
                for m in range(M):
                    hv = gs_b[pl.ds(m * CSZ + o_, LANES)]
                    ob_b[pl.ds(m * CSZ + o_, LANES)] = hv * exs[m // oc]

        def issue_scatters(ci, ob_b, sem):
            for f in range(M + H):
                pltpu.async_copy(
                    ob_b.at[pl.ds(f * CSZ, CSZ)],
                    acc_sh.at[pl.ds(f * N, N)].at[didx.at[ci]], sem,
                    add=True)

        def drain_scatters(ob_b, sem):
            for f in range(M + H):
                pltpu.make_async_copy(
                    ob_b.at[pl.ds(f * CSZ, CSZ)],
                    acc_sh.at[pl.ds(f * N, N)].at[didx.at[0]], sem).wait()

        issue_gathers(0, escA, gsA, gdA, semga)

        def pipe(j, carry):
            c0 = 2 * j
            c1 = 2 * j + 1
            drain_gathers(escA, gsA, gdA, semga)
            issue_gathers(c1, escB, gsB, gdB, semgb)

            @pl.when(j > 0)
            def _():
                drain_scatters(obA, semsa)

            compute(escA, gsA, gdA, obA)
            issue_scatters(c0, obA, semsa)
            drain_gathers(escB, gsB, gdB, semgb)

            @pl.when(j < CH // 2 - 1)
            def _():
                issue_gathers(c0 + 2, escA, gsA, gdA, semga)

            @pl.when(j > 0)
            def _():
                drain_scatters(obB, semsb)

            compute(escB, gsB, gdB, obB)
            issue_scatters(c1, obB, semsb)
            return carry

        lax.fori_loop(0, CH // 2, pipe, 0)
        drain_scatters(obA, semsa)
        drain_scatters(obB, semsb)
        plsc.subcore_barrier()
        pltpu.sync_copy(
            acc_sh.at[pl.ds(s_ * ACC_PER_SUB, ACC_PER_SUB)], stage_v)
        pltpu.sync_copy(
            stage_v,
            out.at[pl.ds(c * (OUTW * N) + s_ * ACC_PER_SUB, ACC_PER_SUB)])

    return pl.kernel(
        body,
        out_type=jax.ShapeDtypeStruct((NC * OUTW * N,), F32),
        mesh=mesh,
        scratch_types=[
            pltpu.VMEM((CH, CSZ), I32),
            pltpu.VMEM((CH, CSZ), I32),
            pltpu.VMEM((CSZ,), F32),
            pltpu.VMEM((CSZ,), F32),
            pltpu.VMEM((SRCW * CSZ,), F32),
            pltpu.VMEM((SRCW * CSZ,), F32),
            pltpu.VMEM((DSTW * CSZ,), F32),
            pltpu.VMEM((DSTW * CSZ,), F32),
            pltpu.VMEM((OUTW * CSZ,), F32),
            pltpu.VMEM((OUTW * CSZ,), F32),
            pltpu.VMEM((ACC_PER_SUB,), F32),
            pltpu.VMEM_SHARED((OUTW * N,), F32),
            pltpu.SemaphoreType.DMA,
            pltpu.SemaphoreType.DMA,
            pltpu.SemaphoreType.DMA,
            pltpu.SemaphoreType.DMA,
        ],
    )


def _comb2_body(p_ref, w2_ref, r1_ref, a2s_ref, a2d_ref,
                x1_ref, tsrc_ref, tdst_ref):
    numT = p_ref[0, 0:30, :] + p_ref[1, 0:30, :]
    denT = p_ref[0, 30:35, :] + p_ref[1, 30:35, :]
    denbT = lax.dot_general(r1_ref[...], denT, (((0,), (0,)), ((), ())),
                            preferred_element_type=F32)
    x1T = numT / (denbT + 1e-16)
    x1_ref[...] = x1T
    h2T = lax.dot_general(w2_ref[...], x1T, (((0,), (0,)), ((), ())),
                          preferred_element_type=F32)
    asrcT = lax.dot_general(a2s_ref[...], h2T, (((1,), (0,)), ((), ())),
                            preferred_element_type=F32)
    adstT = lax.dot_general(a2d_ref[...], h2T, (((1,), (0,)), ((), ())),
                            preferred_element_type=F32)
    n = h2T.shape[1]
    tsrc_ref[...] = jnp.concatenate(
        [h2T, asrcT, jnp.zeros((SRCW - 33, n), F32)], axis=0)
    tdst_ref[...] = jnp.concatenate(
        [adstT, jnp.zeros((DSTW - 1, n), F32)], axis=0)


def _combine_and_tables2(p1, W2, R1, a_src2, a_dst2):
    return pl.pallas_call(
        _comb2_body,
        out_shape=[
            jax.ShapeDtypeStruct((30, N), F32),
            jax.ShapeDtypeStruct((SRCW, N), F32),
            jax.ShapeDtypeStruct((DSTW, N), F32),
        ],
    )(p1, W2, R1, a_src2, a_dst2)


def _softmax32T(p_ref):
    numT = p_ref[0, 0:32, :] + p_ref[1, 0:32, :]
    denT = p_ref[0, 32:33, :] + p_ref[1, 32:33, :]
    sp = numT / (denT + 1e-16)
    m = jnp.max(sp, axis=0, keepdims=True)
    ex = jnp.exp(sp - m)
    return ex / jnp.sum(ex, axis=0, keepdims=True)


BI = 200
NII = N // BI


def _adj_body(p2_ref, adj_ref, y_ref, ssq_ref, s_scr, acc_ref):
    i = pl.program_id(0)

    @pl.when(i == 0)
    def _():
        s_scr[...] = _softmax32T(p2_ref)

    a = adj_ref[...]
    y_ref[...] = lax.dot_general(a, s_scr[...], (((1,), (1,)), ((), ())),
                                 preferred_element_type=F32)

    part = jnp.sum(a * a)

    @pl.when(i == 0)
    def _():
        acc_ref[0, 0] = part

    @pl.when(i != 0)
    def _():
        acc_ref[0, 0] = acc_ref[0, 0] + part

    @pl.when(i == NII - 1)
    def _():
        ssq_ref[0, 0] = acc_ref[0, 0]


def _adj_pass(p2, adj):
    return pl.pallas_call(
        _adj_body,
        grid=(NII,),
        in_specs=[
            pl.BlockSpec((2, OUTW, N), lambda i: (0, 0, 0)),
            pl.BlockSpec((BI, N), lambda i: (i, 0)),
        ],
        out_specs=[
            pl.BlockSpec((BI, 32), lambda i: (i, 0)),
            pl.BlockSpec(memory_space=pltpu.SMEM),
        ],
        out_shape=[
            jax.ShapeDtypeStruct((N, 32), F32),
            jax.ShapeDtypeStruct((1, 1), F32),
        ],
        scratch_shapes=[
            pltpu.VMEM((32, N), F32),
            pltpu.SMEM((1, 1), F32),
        ],
    )(p2, adj)


def _tail_body(p2_ref, y_ref, x1_ref, ssq_ref, w3_ref, a3s_ref, a3d_ref,
               x3_ref, reg_ref):
    sT = _softmax32T(p2_ref)
    cT = (((1,), (1,)), ((), ()))
    c0 = (((0,), (0,)), ((), ()))
    x2 = lax.dot_general(sT, x1_ref[...], cT,
                         preferred_element_type=F32)
    adjn = lax.dot_general(sT, y_ref[...], (((1,), (0,)), ((), ())),
                           preferred_element_type=F32)
    gmat = lax.dot_general(sT, sT, cT,
                           preferred_element_type=F32)
    eye = jnp.where(
        lax.broadcasted_iota(I32, (32, 32), 0)
        == lax.broadcasted_iota(I32, (32, 32), 1),
        jnp.full((32, 32), 1.0, F32), jnp.zeros((32, 32), F32))
    tr1 = jnp.sum(adjn * eye)
    reg1 = jnp.sqrt(ssq_ref[0, 0] - 2.0 * tr1 + jnp.sum(gmat * gmat)) \
        / float(N * N)
    h3 = jnp.dot(x2, w3_ref[...], preferred_element_type=F32)
    av = lax.dot_general(h3, a3s_ref[...], cT,
                         preferred_element_type=F32)
    dv = lax.dot_general(a3d_ref[...], h3, cT,
                         preferred_element_type=F32)
    logit = av + dv + adjn
    logit = jnp.where(logit >= 0, logit, logit * 0.2)
    m0 = jnp.max(logit, axis=0, keepdims=True)
    exl = jnp.exp(logit - m0)
    alpha = exl / (jnp.sum(exl, axis=0, keepdims=True) + 1e-16)
    s2p = lax.dot_general(alpha, h3, c0,
                          preferred_element_type=F32)
    m1 = jnp.max(s2p, axis=-1, keepdims=True)
    e1 = jnp.exp(s2p - m1)
    s2 = e1 / jnp.sum(e1, axis=-1, keepdims=True)
    x3 = lax.dot_general(s2, x2, c0, preferred_element_type=F32)
    sst = lax.dot_general(s2, s2, cT, preferred_element_type=F32)
    reg2 = jnp.sqrt(jnp.sum((adjn - sst) ** 2)) / 1024.0
    x3_ref[...] = x3
    reg_ref[0, 0] = reg1 * 10.0 + reg2 * 0.1


def _tail(p2, y, x1T, ssq, W3, a3s, a3d):
    return pl.pallas_call(
        _tail_body,
        in_specs=[
            pl.BlockSpec((2, OUTW, N), lambda: (0, 0, 0)),
            pl.BlockSpec((N, 32), lambda: (0, 0)),
            pl.BlockSpec((30, N), lambda: (0, 0)),
            pl.BlockSpec(memory_space=pltpu.SMEM),
            pl.BlockSpec((30, 4), lambda: (0, 0)),
            pl.BlockSpec((1, 4), lambda: (0, 0)),
            pl.BlockSpec((1, 4), lambda: (0, 0)),
        ],
        out_specs=[
            pl.BlockSpec((4, 30), lambda: (0, 0)),
            pl.BlockSpec(memory_space=pltpu.SMEM),
        ],
        out_shape=[
            jax.ShapeDtypeStruct((4, 30), F32),
            jax.ShapeDtypeStruct((1, 1), F32),
        ],
    )(p2, y, x1T, ssq, W3, a3s, a3d)


def kernel(x, edge_index, edge_attr, adj, W1, a_src1, a_dst1,
           W2, a_src2, a_dst2, W3, a_src3, a_dst3):
    ei = edge_index.astype(I32)
    src = jnp.pad(ei[0], (0, E_PAD - E))
    dst = jnp.pad(ei[1], (0, E_PAD - E))
    eap = jnp.pad(edge_attr, ((0, E_PAD - E), (0, 0)))
    ea_cols = [eap[:, j].reshape(E_PAD // 128, 128) for j in range(4)]

    As1 = (jnp.eye(5, dtype=F32)[:, None, :]
           * a_src1[:, :, None]).reshape(30, 5)
    Ad1 = (jnp.eye(5, dtype=F32)[:, None, :]
           * a_dst1[:, :, None]).reshape(30, 5)
    R1 = jnp.repeat(jnp.eye(5, dtype=F32), 6, axis=1)
    zrows = jnp.zeros((ACC_PER_SUB,), F32)

    esc2 = _compute_esc(ea_cols)
    src2 = src.reshape(E_PAD // 128, 128)
    dst2 = dst.reshape(E_PAD // 128, 128)
    t1s, t1d = _node_tables1(x, W1, As1, Ad1)
    p1 = _sc_edge_pass(30, 5)(
        t1s.reshape(SRCW * N), t1d.reshape(DSTW * N), src2, dst2, esc2,
        zrows)
    x1T, t2s, t2d = _combine_and_tables2(
        p1.reshape(NC, OUTW, N), W2, R1, a_src2, a_dst2)
    p2 = _sc_edge_pass(32, 1)(
        t2s.reshape(SRCW * N), t2d.reshape(DSTW * N), src2, dst2, esc2,
        zrows)
    p2r = p2.reshape(NC, OUTW, N)
    y, ssq = _adj_pass(p2r, adj)
    x3, reg = _tail(p2r, y, x1T, ssq, W3, a_src3, a_dst3)
    return (x3, reg[0, 0])


# speedup vs baseline: 31.1329x; 1.0303x over previous
import jax
import jax.numpy as jnp
from jax import lax
from jax.experimental import pallas as pl
from jax.experimental.pallas import tpu as pltpu
from jax.experimental.pallas import tpu_sc as plsc

F32 = jnp.float32
I32 = jnp.int32

N = 10000
E = 160000
NC, NS, LANES = 2, 16, 16
NW = NC * NS
CSZ = 64
CH = 80
E_PAD = NW * CSZ * CH
AOSW = 128
AW = 8
OUTW = 128
SUB_ROWS = 624
REM_ROWS = N - SUB_ROWS * NS

_GD = lax.GatherDimensionNumbers(
    offset_dims=(), collapsed_slice_dims=(0,), start_index_map=(0,))


def _splat(vec, i):
    return lax.gather(vec, jnp.full((LANES, 1), i, I32), _GD, (1,),
                      mode=lax.GatherScatterMode.PROMISE_IN_BOUNDS)


def _esc_body(a0, a1, a2, a3, o):
    rows = lax.broadcasted_iota(I32, o.shape, 0)
    cols = lax.broadcasted_iota(I32, o.shape, 1)
    eid = rows * CSZ + cols
    s = a0[...] + a1[...] + a2[...] + a3[...]
    o[...] = jnp.where(eid < E, s, jnp.full_like(s, -1e30))


def _compute_esc(ea_cols):
    shp = ea_cols[0].shape
    return pl.pallas_call(
        _esc_body,
        out_shape=jax.ShapeDtypeStruct(shp, F32),
    )(*ea_cols)


def _node1_body(x_ref, w_ref, As_ref, Ad_ref, taos_ref, tasrc_ref, tdst_ref):
    h = jnp.dot(x_ref[...], w_ref[...], preferred_element_type=F32)
    n = h.shape[0]
    taos_ref[...] = jnp.concatenate(
        [h, jnp.ones((n, 5), F32), jnp.zeros((n, AOSW - 35), F32)], axis=1)
    asrcT = lax.dot_general(As_ref[...], h, (((0,), (1,)), ((), ())),
                            preferred_element_type=F32)
    adstT = lax.dot_general(Ad_ref[...], h, (((0,), (1,)), ((), ())),
                            preferred_element_type=F32)
    z = jnp.zeros((AW - 5, n), F32)
    tasrc_ref[...] = jnp.concatenate([asrcT, z], axis=0)
    tdst_ref[...] = jnp.concatenate([adstT, z], axis=0)


def _node_tables1(x, W1, As, Ad):
    return pl.pallas_call(
        _node1_body,
        out_shape=[
            jax.ShapeDtypeStruct((N, AOSW), F32),
            jax.ShapeDtypeStruct((AW, N), F32),
            jax.ShapeDtypeStruct((AW, N), F32),
        ],
    )(x, W1, As, Ad)


def _sc_edge_pass(M, H):
    mesh = plsc.VectorSubcoreMesh(
        core_axis_name="c", subcore_axis_name="s",
        num_cores=NC, num_subcores=NS)

    def body(taos, tasrc, tadst, src2, dst2, esc2, zrows, out,
             sidx, didx, escA, escB, gaA, gaB, gdA, gdB, gsA, gsB,
             ob, acc_sh, semga, semgb, sems):
        c = lax.axis_index("c")
        s_ = lax.axis_index("s")
        tid = c * NS + s_
        pltpu.sync_copy(src2.at[pl.ds(tid * CH * CSZ, CH * CSZ)], sidx)
        pltpu.sync_copy(dst2.at[pl.ds(tid * (CH // 2), CH // 2)], didx)
        pltpu.sync_copy(zrows, acc_sh.at[pl.ds(s_ * SUB_ROWS, SUB_ROWS)])

        @pl.when(s_ == NS - 1)
        def _():
            pltpu.sync_copy(zrows.at[pl.ds(0, REM_ROWS)],
                            acc_sh.at[pl.ds(NS * SUB_ROWS, REM_ROWS)])

        plsc.subcore_barrier()

        def issue_gathers(ci, esc_b, ga_b, gd_b, gs_b, sem):
            pltpu.async_copy(esc2.at[tid * CH + ci], esc_b, sem)
            srow = sidx.at[pl.ds(ci * CSZ, CSZ)]
            drow = didx.at[ci // 2, pl.ds((ci % 2) * CSZ, CSZ)]
            for f in range(H):
                pltpu.async_copy(
                    tasrc.at[pl.ds(f * N, N)].at[srow],
                    ga_b.at[pl.ds(f * CSZ, CSZ)], sem)
                pltpu.async_copy(
                    tadst.at[pl.ds(f * N, N)].at[drow],
                    gd_b.at[pl.ds(f * CSZ, CSZ)], sem)
            pltpu.async_copy(taos.at[srow], gs_b, sem)

        def drain_gathers(esc_b, ga_b, gd_b, gs_b, sem):
            pltpu.make_async_copy(esc2.at[0], esc_b, sem).wait()
            srow0 = sidx.at[pl.ds(0, CSZ)]
            for f in range(H):
                pltpu.make_async_copy(
                    tasrc.at[pl.ds(f * N, N)].at[srow0],
                    ga_b.at[pl.ds(f * CSZ, CSZ)], sem).wait()
                pltpu.make_async_copy(
                    tadst.at[pl.ds(f * N, N)].at[srow0],
                    gd_b.at[pl.ds(f * CSZ, CSZ)], sem).wait()
            pltpu.make_async_copy(taos.at[srow0], gs_b, sem).wait()

        def compute(esc_b, ga_b, gd_b, gs_b, half):
            lane = lax.iota(I32, LANES)
            for g in range(CSZ // LANES):
                o_ = g * LANES
                ev = esc_b[pl.ds(o_, LANES)]
                exs = []
                for h in range(H):
                    l = (ga_b[pl.ds(h * CSZ + o_, LANES)]
                         + gd_b[pl.ds(h * CSZ + o_, LANES)] + ev)
                    l = jnp.where(l >= 0, l, l * 0.2)
                    exs.append(jnp.exp(l))
                for i in range(LANES):
                    e = half * CSZ + o_ + i
                    if H == 1:
                        spl = _splat(exs[0], i)
                        c0 = c1 = c2 = spl
                    else:
                        s0 = _splat(exs[0], i)
                        s1 = _splat(exs[1], i)
                        s2 = _splat(exs[2], i)
                        s3 = _splat(exs[3], i)
                        s4 = _splat(exs[4], i)
                        c0 = jnp.where(lane < 6, s0,
                                       jnp.where(lane < 12, s1, s2))
                        c1 = jnp.where(
                            lane < 2, s2,
                            jnp.where(lane < 8, s3,
                                      jnp.where(lane < 14, s4,
                                                jnp.where(lane < 15, s0,
                                                          s1))))
                        c2 = jnp.where(lane == 0, s2,
                                       jnp.where(lane == 1, s3, s4))
                    ob[e, pl.ds(0, LANES)] = \
                        gs_b[o_ + i, pl.ds(0, LANES)] * c0
                    ob[e, pl.ds(LANES, LANES)] = \
                        gs_b[o_ + i, pl.ds(LANES, LANES)] * c1
                    ob[e, pl.ds(2 * LANES, LANES)] = \
                        gs_b[o_ + i, pl.ds(2 * LANES, LANES)] * c2

        def issue_scatter(p):
            pltpu.async_copy(ob, acc_sh.at[didx.at[p]], sems, add=True)

        def drain_scatter():
            pltpu.make_async_copy(ob, acc_sh.at[didx.at[0]], sems).wait()

        issue_gathers(0, escA, gaA, gdA, gsA, semga)

        def pipe(p, carry):
            c0_ = 2 * p
            c1_ = 2 * p + 1
            drain_gathers(escA, gaA, gdA, gsA, semga)
            issue_gathers(c1_, escB, gaB, gdB, gsB, semgb)

            @pl.when(p > 0)
            def _():
                drain_scatter()

            compute(escA, gaA, gdA, gsA, 0)
            drain_gathers(escB, gaB, gdB, gsB, semgb)

            @pl.when(p < CH // 2 - 1)
            def _():
                issue_gathers(c0_ + 2, escA, gaA, gdA, gsA, semga)

            compute(escB, gaB, gdB, gsB, 1)
            issue_scatter(p)
            return carry

        lax.fori_loop(0, CH // 2, pipe, 0)
        drain_scatter()
        plsc.subcore_barrier()
        pltpu.sync_copy(acc_sh.at[pl.ds(s_ * SUB_ROWS, SUB_ROWS)],
                        out.at[c, pl.ds(s_ * SUB_ROWS, SUB_ROWS)])

        @pl.when(s_ == NS - 1)
        def _():
            pltpu.sync_copy(acc_sh.at[pl.ds(NS * SUB_ROWS, REM_ROWS)],
                            out.at[c, pl.ds(NS * SUB_ROWS, REM_ROWS)])

    return pl.kernel(
        body,
        out_type=jax.ShapeDtypeStruct((NC, N, OUTW), F32),
        mesh=mesh,
        scratch_types=[
            pltpu.VMEM((CH * CSZ,), I32),
            pltpu.VMEM((CH // 2, 2 * CSZ), I32),
            pltpu.VMEM((CSZ,), F32),
            pltpu.VMEM((CSZ,), F32),
            pltpu.VMEM((AW * CSZ,), F32),
            pltpu.VMEM((AW * CSZ,), F32),
            pltpu.VMEM((AW * CSZ,), F32),
            pltpu.VMEM((AW * CSZ,), F32),
            pltpu.VMEM((CSZ, AOSW), F32),
            pltpu.VMEM((CSZ, AOSW), F32),
            pltpu.VMEM((2 * CSZ, OUTW), F32),
            pltpu.VMEM_SHARED((N, OUTW), F32),
            pltpu.SemaphoreType.DMA,
            pltpu.SemaphoreType.DMA,
            pltpu.SemaphoreType.DMA,
        ],
    )


def _comb2_body(p_ref, w2_ref, r1_ref, a2s_ref, a2d_ref,
                x1_ref, taos_ref, tasrc_ref, tdst_ref):
    num = p_ref[0, :, 0:30] + p_ref[1, :, 0:30]
    den = p_ref[0, :, 30:35] + p_ref[1, :, 30:35]
    denb = jnp.dot(den, r1_ref[...], preferred_element_type=F32)
    x1 = num / (denb + 1e-16)
    x1_ref[...] = x1
    h2 = jnp.dot(x1, w2_ref[...], preferred_element_type=F32)
    n = h2.shape[0]
    taos_ref[...] = jnp.concatenate(
        [h2, jnp.ones((n, 1), F32), jnp.zeros((n, AOSW - 33), F32)], axis=1)
    asrcT = lax.dot_general(a2s_ref[...], h2, (((1,), (1,)), ((), ())),
                            preferred_element_type=F32)
    adstT = lax.dot_general(a2d_ref[...], h2, (((1,), (1,)), ((), ())),
                            preferred_element_type=F32)
    z = jnp.zeros((AW - 1, n), F32)
    tasrc_ref[...] = jnp.concatenate([asrcT, z], axis=0)
    tdst_ref[...] = jnp.concatenate([adstT, z], axis=0)


def _combine_and_tables2(p1, W2, R1, a_src2, a_dst2):
    return pl.pallas_call(
        _comb2_body,
        out_shape=[
            jax.ShapeDtypeStruct((N, 30), F32),
            jax.ShapeDtypeStruct((N, AOSW), F32),
            jax.ShapeDtypeStruct((AW, N), F32),
            jax.ShapeDtypeStruct((AW, N), F32),
        ],
    )(p1, W2, R1, a_src2, a_dst2)


def _softmax32(p_ref):
    num = p_ref[0, :, 0:32] + p_ref[1, :, 0:32]
    den = p_ref[0, :, 32:33] + p_ref[1, :, 32:33]
    sp = num / (den + 1e-16)
    m = jnp.max(sp, axis=-1, keepdims=True)
    ex = jnp.exp(sp - m)
    return ex / jnp.sum(ex, axis=-1, keepdims=True)


BI = 200
NII = N // BI


def _adj_body(p2_ref, adj_ref, y_ref, ssq_ref, s_scr, acc_ref):
    i = pl.program_id(0)

    @pl.when(i == 0)
    def _():
        s_scr[...] = _softmax32(p2_ref)

    a = adj_ref[...]
    y_ref[...] = jnp.dot(a, s_scr[...], preferred_element_type=F32)

    part = jnp.sum(a * a)

    @pl.when(i == 0)
    def _():
        acc_ref[0, 0] = part

    @pl.when(i != 0)
    def _():
        acc_ref[0, 0] = acc_ref[0, 0] + part

    @pl.when(i == NII - 1)
    def _():
        ssq_ref[0, 0] = acc_ref[0, 0]


def _adj_pass(p2, adj):
    return pl.pallas_call(
        _adj_body,
        grid=(NII,),
        in_specs=[
            pl.BlockSpec((2, N, OUTW), lambda i: (0, 0, 0)),
            pl.BlockSpec((BI, N), lambda i: (i, 0)),
        ],
        out_specs=[
            pl.BlockSpec((BI, 32), lambda i: (i, 0)),
            pl.BlockSpec(memory_space=pltpu.SMEM),
        ],
        out_shape=[
            jax.ShapeDtypeStruct((N, 32), F32),
            jax.ShapeDtypeStruct((1, 1), F32),
        ],
        scratch_shapes=[
            pltpu.VMEM((N, 32), F32),
            pltpu.SMEM((1, 1), F32),
        ],
    )(p2, adj)


def _tail_body(p2_ref, y_ref, x1_ref, ssq_ref, w3_ref, a3s_ref, a3d_ref,
               x3_ref, reg_ref):
    s = _softmax32(p2_ref)
    c0 = (((0,), (0,)), ((), ()))
    cT = (((1,), (1,)), ((), ()))
    x2 = lax.dot_general(s, x1_ref[...], c0,
                         preferred_element_type=F32)
    adjn = lax.dot_general(s, y_ref[...], c0,
                           preferred_element_type=F32)
    gmat = lax.dot_general(s, s, c0,
                           preferred_element_type=F32)
    eye = jnp.where(
        lax.broadcasted_iota(I32, (32, 32), 0)
        == lax.broadcasted_iota(I32, (32, 32), 1),
        jnp.full((32, 32), 1.0, F32), jnp.zeros((32, 32), F32))
    tr1 = jnp.sum(adjn * eye)
    reg1 = jnp.sqrt(ssq_ref[0, 0] - 2.0 * tr1 + jnp.sum(gmat * gmat)) \
        / float(N * N)
    h3 = jnp.dot(x2, w3_ref[...], preferred_element_type=F32)
    av = lax.dot_general(h3, a3s_ref[...], cT,
                         preferred_element_type=F32)
    dv = lax.dot_general(a3d_ref[...], h3, cT,
                         preferred_element_type=F32)
    logit = av + dv + adjn
    logit = jnp.where(logit >= 0, logit, logit * 0.2)
    m0 = jnp.max(logit, axis=0, keepdims=True)
    exl = jnp.exp(logit - m0)
    alpha = exl / (jnp.sum(exl, axis=0, keepdims=True) + 1e-16)
    s2p = lax.dot_general(alpha, h3, c0,
                          preferred_element_type=F32)
    m1 = jnp.max(s2p, axis=-1, keepdims=True)
    e1 = jnp.exp(s2p - m1)
    s2 = e1 / jnp.sum(e1, axis=-1, keepdims=True)
    x3 = lax.dot_general(s2, x2, c0, preferred_element_type=F32)
    sst = lax.dot_general(s2, s2, cT, preferred_element_type=F32)
    reg2 = jnp.sqrt(jnp.sum((adjn - sst) ** 2)) / 1024.0
    x3_ref[...] = x3
    reg_ref[0, 0] = reg1 * 10.0 + reg2 * 0.1


def _tail(p2, y, x1, ssq, W3, a3s, a3d):
    return pl.pallas_call(
        _tail_body,
        in_specs=[
            pl.BlockSpec((2, N, OUTW), lambda: (0, 0, 0)),
            pl.BlockSpec((N, 32), lambda: (0, 0)),
            pl.BlockSpec((N, 30), lambda: (0, 0)),
            pl.BlockSpec(memory_space=pltpu.SMEM),
            pl.BlockSpec((30, 4), lambda: (0, 0)),
            pl.BlockSpec((1, 4), lambda: (0, 0)),
            pl.BlockSpec((1, 4), lambda: (0, 0)),
        ],
        out_specs=[
            pl.BlockSpec((4, 30), lambda: (0, 0)),
            pl.BlockSpec(memory_space=pltpu.SMEM),
        ],
        out_shape=[
            jax.ShapeDtypeStruct((4, 30), F32),
            jax.ShapeDtypeStruct((1, 1), F32),
        ],
    )(p2, y, x1, ssq, W3, a3s, a3d)


def kernel(x, edge_index, edge_attr, adj, W1, a_src1, a_dst1,
           W2, a_src2, a_dst2, W3, a_src3, a_dst3):
    ei = edge_index.astype(I32)
    src2 = jnp.pad(ei[0], (0, E_PAD - E))
    dst2 = jnp.pad(ei[1], (0, E_PAD - E)).reshape(E_PAD // (2 * CSZ),
                                                  2 * CSZ)
    eap = jnp.pad(edge_attr, ((0, E_PAD - E), (0, 0)))
    ea_cols = [eap[:, j].reshape(E_PAD // CSZ, CSZ) for j in range(4)]

    As1 = (jnp.eye(5, dtype=F32)[:, None, :]
           * a_src1[:, :, None]).reshape(30, 5)
    Ad1 = (jnp.eye(5, dtype=F32)[:, None, :]
           * a_dst1[:, :, None]).reshape(30, 5)
    R1 = jnp.repeat(jnp.eye(5, dtype=F32), 6, axis=1)
    zrows = jnp.zeros((SUB_ROWS, OUTW), F32)

    esc2 = _compute_esc(ea_cols)
    taos1, tasrc1, tdst1 = _node_tables1(x, W1, As1, Ad1)
    p1 = _sc_edge_pass(30, 5)(
        taos1, tasrc1.reshape(AW * N), tdst1.reshape(AW * N),
        src2, dst2, esc2, zrows)
    x1, taos2, tasrc2, tdst2 = _combine_and_tables2(
        p1, W2, R1, a_src2, a_dst2)
    p2 = _sc_edge_pass(32, 1)(
        taos2, tasrc2.reshape(AW * N), tdst2.reshape(AW * N),
        src2, dst2, esc2, zrows)
    y, ssq = _adj_pass(p2, adj)
    x3, reg = _tail(p2, y, x1, ssq, W3, a_src3, a_dst3)
    return (x3, reg[0, 0])
